# Initial kernel scaffold; baseline (speedup 1.0000x reference)
import jax
import jax.numpy as jnp
from jax.experimental import pallas as pl


def kernel(x, W1, b1, W2, b2):
    raise NotImplementedError("write your pallas kernel here")


# speedup vs baseline: 2.3751x; 2.3751x over previous
import functools

import jax
import jax.numpy as jnp
from jax import lax
from jax.experimental import pallas as pl
from jax.experimental.pallas import tpu as pltpu

K = 8
BT = 256


def _router_body(x_ref, W1_ref, b1_ref, W2_ref, b2_ref, probs_ref, idx_ref):
    bt, E = probs_ref.shape
    x = x_ref[...]
    h = lax.dot_general(x, W1_ref[...], (((1,), (1,)), ((), ())),
                        preferred_element_type=jnp.float32)
    h = jnp.maximum(h + b1_ref[...], 0.0)
    logits = lax.dot_general(h, W2_ref[...], (((1,), (1,)), ((), ())),
                             preferred_element_type=jnp.float32)
    logits = logits + b2_ref[...]

    iota = lax.broadcasted_iota(jnp.int32, (bt, E), 1)
    neg_inf = jnp.float32(-jnp.inf)
    work = logits
    row_max = None
    for k in range(K):
        m = jnp.max(work, axis=-1, keepdims=True)
        if k == 0:
            row_max = m
        idx_k = jnp.min(jnp.where(work == m, iota, E), axis=-1,
                        keepdims=True)
        work = jnp.where(iota == idx_k, neg_inf, work)
        idx_ref[:, k:k + 1] = idx_k
    kept = work == neg_inf
    e = jnp.where(kept, jnp.exp(logits - row_max), 0.0)
    probs_ref[...] = e / jnp.sum(e, axis=-1, keepdims=True)


def kernel(x, W1, b1, W2, b2):
    T, D = x.shape
    H = W1.shape[0]
    E = W2.shape[0]
    grid = (T // BT,)
    probs, idx = pl.pallas_call(
        _router_body,
        grid=grid,
        in_specs=[
            pl.BlockSpec((BT, D), lambda i: (i, 0)),
            pl.BlockSpec((H, D), lambda i: (0, 0)),
            pl.BlockSpec((1, H), lambda i: (0, 0)),
            pl.BlockSpec((E, H), lambda i: (0, 0)),
            pl.BlockSpec((1, E), lambda i: (0, 0)),
        ],
        out_specs=[
            pl.BlockSpec((BT, E), lambda i: (i, 0)),
            pl.BlockSpec((BT, K), lambda i: (i, 0)),
        ],
        out_shape=[
            jax.ShapeDtypeStruct((T, E), jnp.float32),
            jax.ShapeDtypeStruct((T, K), jnp.int32),
        ],
    )(x, W1, b1.reshape(1, H), W2, b2.reshape(1, E))
    return (probs, idx)


# speedup vs baseline: 2.9731x; 1.2518x over previous
import functools

import jax
import jax.numpy as jnp
from jax import lax
from jax.experimental import pallas as pl
from jax.experimental.pallas import tpu as pltpu

K = 8
BT = 512


def _router_body(x_ref, W1_ref, b1_ref, W2_ref, b2_ref, probs_ref, idx_ref):
    bt, E = probs_ref.shape
    x = x_ref[...]
    h = lax.dot_general(x, W1_ref[...], (((1,), (1,)), ((), ())),
                        preferred_element_type=jnp.float32)
    h = jnp.maximum(h + b1_ref[...], 0.0)
    logits = lax.dot_general(h, W2_ref[...], (((1,), (1,)), ((), ())),
                             preferred_element_type=jnp.float32)
    logits = logits + b2_ref[...]

    iota_f = lax.broadcasted_iota(jnp.int32, (bt, E), 1).astype(jnp.float32)
    neg_inf = jnp.float32(-jnp.inf)
    big = jnp.float32(E)
    work = logits
    row_max = None
    idx_cols = []
    for k in range(K):
        m = jnp.max(work, axis=-1, keepdims=True)
        if k == 0:
            row_max = m
        idx_k = jnp.min(jnp.where(work == m, iota_f, big), axis=-1,
                        keepdims=True)
        work = jnp.where(iota_f == idx_k, neg_inf, work)
        idx_cols.append(idx_k)
    idx_ref[...] = jnp.concatenate(idx_cols, axis=1).astype(jnp.int32)
    kept = work == neg_inf
    e = jnp.where(kept, jnp.exp(logits - row_max), 0.0)
    probs_ref[...] = e / jnp.sum(e, axis=-1, keepdims=True)


def kernel(x, W1, b1, W2, b2):
    T, D = x.shape
    H = W1.shape[0]
    E = W2.shape[0]
    grid = (T // BT,)
    probs, idx = pl.pallas_call(
        _router_body,
        grid=grid,
        in_specs=[
            pl.BlockSpec((BT, D), lambda i: (i, 0)),
            pl.BlockSpec((H, D), lambda i: (0, 0)),
            pl.BlockSpec((1, H), lambda i: (0, 0)),
            pl.BlockSpec((E, H), lambda i: (0, 0)),
            pl.BlockSpec((1, E), lambda i: (0, 0)),
        ],
        out_specs=[
            pl.BlockSpec((BT, E), lambda i: (i, 0)),
            pl.BlockSpec((BT, K), lambda i: (i, 0)),
        ],
        out_shape=[
            jax.ShapeDtypeStruct((T, E), jnp.float32),
            jax.ShapeDtypeStruct((T, K), jnp.int32),
        ],
    )(x, W1, b1.reshape(1, H), W2, b2.reshape(1, E))
    return (probs, idx)
